# Initial kernel scaffold; baseline (speedup 1.0000x reference)
import functools
import jax
import jax.numpy as jnp
from jax import lax
from jax.experimental import pallas as pl
from jax.experimental.pallas import tpu as pltpu
from jax.experimental.pallas import tpu_sc as plsc

D = 32
B_TOTAL = 4096 * 200
NC, NS = 2, 16
NW = NC * NS
B_PER_W = B_TOTAL // NW
CHUNK = 128
N_CHUNKS = B_PER_W // CHUNK

_mesh = plsc.VectorSubcoreMesh(core_axis_name="c", subcore_axis_name="s")


@functools.partial(
    pl.kernel,
    mesh=_mesh,
    out_type=jax.ShapeDtypeStruct((B_TOTAL, D), jnp.float32),
    scratch_types=[
        pltpu.VMEM((N_CHUNKS, CHUNK), jnp.int32),
        pltpu.VMEM((CHUNK, D), jnp.float32),
        pltpu.SemaphoreType.DMA,
    ],
)
def _gather_kernel(table_hbm, idx_hbm, out_hbm, idx_v, rows_v, sem):
    w = lax.axis_index("s") * NC + lax.axis_index("c")
    base = w * B_PER_W
    pltpu.sync_copy(idx_hbm.at[w], idx_v)

    def step(j, carry):
        pltpu.async_copy(table_hbm.at[idx_v.at[j]], rows_v, sem).wait()
        pltpu.sync_copy(rows_v, out_hbm.at[pl.ds(base + j * CHUNK, CHUNK)])
        return carry

    lax.fori_loop(0, N_CHUNKS, step, 0)


def kernel(batch, weight):
    idx = batch.reshape(-1).astype(jnp.int32).reshape(NW, N_CHUNKS, CHUNK)
    out = _gather_kernel(weight, idx)
    return out.reshape(batch.shape + (D,))


# speedup vs baseline: 1.3070x; 1.3070x over previous
import functools
import jax
import jax.numpy as jnp
from jax import lax
from jax.experimental import pallas as pl
from jax.experimental.pallas import tpu as pltpu
from jax.experimental.pallas import tpu_sc as plsc

D = 32
B_TOTAL = 4096 * 200
NC, NS = 2, 16
NW = NC * NS
B_PER_W = B_TOTAL // NW
CHUNK = 128
N_CHUNKS = B_PER_W // CHUNK

_mesh = plsc.VectorSubcoreMesh(core_axis_name="c", subcore_axis_name="s")


@functools.partial(
    pl.kernel,
    mesh=_mesh,
    out_type=jax.ShapeDtypeStruct((B_TOTAL, D), jnp.float32),
    scratch_types=[
        pltpu.VMEM((N_CHUNKS, CHUNK), jnp.int32),
        pltpu.VMEM((CHUNK, D), jnp.float32),
        pltpu.SemaphoreType.DMA,
    ],
    compiler_params=pltpu.CompilerParams(use_tc_tiling_on_sc=False),
)
def _gather_kernel(table_hbm, idx_hbm, out_hbm, idx_v, rows_v, sem):
    w = lax.axis_index("s") * NC + lax.axis_index("c")
    base = w * B_PER_W
    pltpu.sync_copy(idx_hbm.at[w], idx_v)

    def step(j, carry):
        pltpu.async_copy(table_hbm.at[idx_v.at[j]], rows_v, sem).wait()
        pltpu.sync_copy(rows_v, out_hbm.at[pl.ds(base + j * CHUNK, CHUNK)])
        return carry

    lax.fori_loop(0, N_CHUNKS, step, 0)


def kernel(batch, weight):
    idx = batch.reshape(-1).astype(jnp.int32).reshape(NW, N_CHUNKS, CHUNK)
    out = _gather_kernel(weight, idx)
    return out.reshape(batch.shape + (D,))


# speedup vs baseline: 1.5012x; 1.1485x over previous
import functools
import jax
import jax.numpy as jnp
from jax import lax
from jax.experimental import pallas as pl
from jax.experimental.pallas import tpu as pltpu
from jax.experimental.pallas import tpu_sc as plsc

D = 32
B_TOTAL = 4096 * 200
NC, NS = 2, 16
NW = NC * NS
B_PER_W = B_TOTAL // NW
CHUNK = 128
N_CHUNKS = B_PER_W // CHUNK
K = 8
GROUP = K * CHUNK
NG = N_CHUNKS // K

_mesh = plsc.VectorSubcoreMesh(core_axis_name="c", subcore_axis_name="s")


@functools.partial(
    pl.kernel,
    mesh=_mesh,
    out_type=jax.ShapeDtypeStruct((B_TOTAL, D), jnp.float32),
    scratch_types=[
        pltpu.VMEM((N_CHUNKS, CHUNK), jnp.int32),
        pltpu.VMEM((GROUP, D), jnp.float32),
        pltpu.VMEM((GROUP, D), jnp.float32),
        pltpu.SemaphoreType.DMA,
        pltpu.SemaphoreType.DMA,
    ],
    compiler_params=pltpu.CompilerParams(use_tc_tiling_on_sc=False),
)
def _gather_kernel(table_hbm, idx_hbm, out_hbm, idx_v, rows0, rows1, sem0, sem1):
    w = lax.axis_index("s") * NC + lax.axis_index("c")
    base = w * B_PER_W
    pltpu.sync_copy(idx_hbm.at[w], idx_v)

    bufs = (rows0, rows1)
    sems = (sem0, sem1)

    def fire(g, buf, sem):
        for k in range(K):
            pltpu.async_copy(table_hbm.at[idx_v.at[g * K + k]],
                             buf.at[pl.ds(k * CHUNK, CHUNK)], sem)

    def drain(g, buf, sem):
        for k in range(K):
            pltpu.make_async_copy(table_hbm.at[idx_v.at[g * K + k]],
                                  buf.at[pl.ds(k * CHUNK, CHUNK)], sem).wait()

    fire(0, rows0, sem0)
    fire(1, rows1, sem1)

    def step(i, carry):
        for b in range(2):
            g = 2 * i + b

            @pl.when(g < NG)
            def _():
                drain(g, bufs[b], sems[b])
                pltpu.sync_copy(bufs[b],
                                out_hbm.at[pl.ds(base + g * GROUP, GROUP)])

                @pl.when(g + 2 < NG)
                def _():
                    fire(g + 2, bufs[b], sems[b])
        return carry

    lax.fori_loop(0, (NG + 1) // 2, step, 0, unroll=False)


def kernel(batch, weight):
    idx = batch.reshape(-1).astype(jnp.int32).reshape(NW, N_CHUNKS, CHUNK)
    out = _gather_kernel(weight, idx)
    return out.reshape(batch.shape + (D,))


# speedup vs baseline: 4.4248x; 2.9475x over previous
import functools
import jax
import jax.numpy as jnp
from jax import lax
from jax.experimental import pallas as pl
from jax.experimental.pallas import tpu as pltpu
from jax.experimental.pallas import tpu_sc as plsc

V = 1000000
D = 32
B = 4096
T = 200
NC, NS = 2, 16
NW = NC * NS
CHUNK = 128
NBUF = 6
AHEAD = 3

VP = 1 << 18
W1 = 16384
NLB = (V + W1 - 1) // W1


def _t1_body(x0, x1, x2, x3, o_ref):
    z = jnp.concatenate([x0[...], x1[...], x2[...], x3[...]], axis=0)
    o_ref[...] = z.T


def _t1_in_spec(p):
    return pl.BlockSpec(
        (D, W1), lambda i, p=p: (0, jnp.minimum(4 * i + p, NLB - 1)))


_t1 = pl.pallas_call(
    _t1_body,
    grid=(16,),
    in_specs=[_t1_in_spec(p) for p in range(4)],
    out_specs=pl.BlockSpec((W1, 128), lambda i: (i, 0)),
    out_shape=jax.ShapeDtypeStruct((W1 * 16, 128), jnp.float32),
)


def _t2_body(x_ref, o_ref):
    z = x_ref[0].T
    for u in range(4):
        o_ref[0, :, 1024 * u:1024 * (u + 1)] = z[32 * u:32 * (u + 1), :]


_t2 = pl.pallas_call(
    _t2_body,
    grid=(T,),
    in_specs=[pl.BlockSpec((1, B // 4, 128), lambda i: (i, 0, 0))],
    out_specs=pl.BlockSpec((1, D, B), lambda i: (i, 0, 0)),
    out_shape=jax.ShapeDtypeStruct((T, D, B), jnp.float32),
)

_mesh = plsc.VectorSubcoreMesh(core_axis_name="c", subcore_axis_name="s")


@functools.partial(
    pl.kernel,
    mesh=_mesh,
    out_type=jax.ShapeDtypeStruct((B * T // 4, 4, D), jnp.float32),
    scratch_types=(
        [pltpu.VMEM((T, CHUNK), jnp.int32)]
        + [pltpu.VMEM((CHUNK, D), jnp.float32) for _ in range(NBUF)]
        + [pltpu.SemaphoreType.DMA for _ in range(2 * NBUF)]
    ),
    compiler_params=pltpu.CompilerParams(use_tc_tiling_on_sc=False),
)
def _gather_kernel(table_hbm, idx_hbm, out_hbm, idx_v, *scr):
    w = lax.axis_index("s") * NC + lax.axis_index("c")
    bufs = scr[:NBUF]
    gsems = scr[NBUF:2 * NBUF]
    wsems = scr[2 * NBUF:]
    pltpu.sync_copy(idx_hbm.at[w], idx_v)

    u = w // 8
    m0 = CHUNK * (w % 8)

    def out_slice(t):
        return out_hbm.at[pl.ds(t * (B // 4) + m0, CHUNK), u]

    def fire_gather(t, b):
        pltpu.async_copy(table_hbm.at[idx_v.at[t]], bufs[b], gsems[b])

    for b in range(AHEAD):
        fire_gather(b, b)

    def step(i, carry):
        for b in range(NBUF):
            t = NBUF * i + b

            @pl.when(t < T)
            def _():
                tn = t + AHEAD
                bn = (b + AHEAD) % NBUF

                @pl.when(tn < T)
                def _():
                    @pl.when(tn >= NBUF)
                    def _():
                        pltpu.make_async_copy(bufs[bn], out_slice(0),
                                              wsems[bn]).wait()
                    fire_gather(tn, bn)

                pltpu.make_async_copy(table_hbm.at[idx_v.at[t]], bufs[b],
                                      gsems[b]).wait()
                pltpu.async_copy(bufs[b], out_slice(t), wsems[b])
        return carry

    lax.fori_loop(0, (T + NBUF - 1) // NBUF, step, 0, unroll=False)

    for b in range(NBUF):
        pltpu.make_async_copy(bufs[b], out_slice(0), wsems[b]).wait()


def kernel(batch, weight):
    wt = weight.T
    table = _t1(wt, wt, wt, wt)
    table = table.reshape(W1 * 16 * 4, D)
    r = batch.astype(jnp.int32).reshape(NW, CHUNK, T).transpose(0, 2, 1)
    idx3 = (r & ~0xFFFF) | ((r & 0x3FFF) << 2) | ((r >> 14) & 3)
    flat3 = _gather_kernel(table, idx3)
    out_t = _t2(flat3.reshape(T, B // 4, 128))
    return out_t.transpose(2, 0, 1)


# speedup vs baseline: 6.0004x; 1.3561x over previous
import functools
import jax
import jax.numpy as jnp
from jax import lax
from jax.experimental import pallas as pl
from jax.experimental.pallas import tpu as pltpu
from jax.experimental.pallas import tpu_sc as plsc

V = 1000000
D = 32
B = 4096
T = 200
NC, NS = 2, 16
NW = NC * NS
CHUNK = 128
NBUF = 6
AHEAD = 3

VP = 1 << 18
W1 = 16384
NLB = (V + W1 - 1) // W1


def _t1_body(x0, x1, x2, x3, o_ref):
    z = jnp.concatenate([x0[...], x1[...], x2[...], x3[...]], axis=0)
    o_ref[...] = z.T


def _t1_in_spec(p):
    return pl.BlockSpec(
        (D, W1), lambda i, p=p: (0, jnp.minimum(4 * i + p, NLB - 1)))


_t1 = pl.pallas_call(
    _t1_body,
    grid=(16,),
    in_specs=[_t1_in_spec(p) for p in range(4)],
    out_specs=pl.BlockSpec((W1, 128), lambda i: (i, 0)),
    out_shape=jax.ShapeDtypeStruct((W1 * 16, 128), jnp.float32),
)


T2_TB = 8


def _t2_body(x_ref, o_ref):
    for tt in range(T2_TB):
        z = x_ref[tt].T
        for u in range(4):
            o_ref[tt, :, 1024 * u:1024 * (u + 1)] = z[32 * u:32 * (u + 1), :]


_t2 = pl.pallas_call(
    _t2_body,
    grid=(T // T2_TB,),
    in_specs=[pl.BlockSpec((T2_TB, B // 4, 128), lambda i: (i, 0, 0))],
    out_specs=pl.BlockSpec((T2_TB, D, B), lambda i: (i, 0, 0)),
    out_shape=jax.ShapeDtypeStruct((T, D, B), jnp.float32),
)

_mesh = plsc.VectorSubcoreMesh(core_axis_name="c", subcore_axis_name="s")


@functools.partial(
    pl.kernel,
    mesh=_mesh,
    out_type=jax.ShapeDtypeStruct((B * T // 4, 4, D), jnp.float32),
    scratch_types=(
        [pltpu.VMEM((T, CHUNK), jnp.int32)]
        + [pltpu.VMEM((CHUNK, D), jnp.float32) for _ in range(NBUF)]
        + [pltpu.SemaphoreType.DMA for _ in range(2 * NBUF)]
    ),
    compiler_params=pltpu.CompilerParams(use_tc_tiling_on_sc=False),
)
def _gather_kernel(table_hbm, idx_hbm, out_hbm, idx_v, *scr):
    w = lax.axis_index("s") * NC + lax.axis_index("c")
    bufs = scr[:NBUF]
    gsems = scr[NBUF:2 * NBUF]
    wsems = scr[2 * NBUF:]
    pltpu.sync_copy(idx_hbm.at[w], idx_v)

    u = w // 8
    m0 = CHUNK * (w % 8)

    def out_slice(t):
        return out_hbm.at[pl.ds(t * (B // 4) + m0, CHUNK), u]

    def fire_gather(t, b):
        pltpu.async_copy(table_hbm.at[idx_v.at[t]], bufs[b], gsems[b])

    for b in range(AHEAD):
        fire_gather(b, b)

    def step(i, carry):
        for b in range(NBUF):
            t = NBUF * i + b

            @pl.when(t < T)
            def _():
                tn = t + AHEAD
                bn = (b + AHEAD) % NBUF

                @pl.when(tn < T)
                def _():
                    @pl.when(tn >= NBUF)
                    def _():
                        pltpu.make_async_copy(bufs[bn], out_slice(0),
                                              wsems[bn]).wait()
                    fire_gather(tn, bn)

                pltpu.make_async_copy(table_hbm.at[idx_v.at[t]], bufs[b],
                                      gsems[b]).wait()
                pltpu.async_copy(bufs[b], out_slice(t), wsems[b])
        return carry

    lax.fori_loop(0, (T + NBUF - 1) // NBUF, step, 0, unroll=False)

    for b in range(NBUF):
        pltpu.make_async_copy(bufs[b], out_slice(0), wsems[b]).wait()


def kernel(batch, weight):
    wt = weight.T
    table = _t1(wt, wt, wt, wt)
    table = table.reshape(W1 * 16 * 4, D)
    r = batch.astype(jnp.int32).reshape(NW, CHUNK, T).transpose(0, 2, 1)
    idx3 = (r & ~0xFFFF) | ((r & 0x3FFF) << 2) | ((r >> 14) & 3)
    flat3 = _gather_kernel(table, idx3)
    out_t = _t2(flat3.reshape(T, B // 4, 128))
    return out_t.transpose(2, 0, 1)


# speedup vs baseline: 6.1350x; 1.0224x over previous
import functools
import jax
import jax.numpy as jnp
from jax import lax
from jax.experimental import pallas as pl
from jax.experimental.pallas import tpu as pltpu
from jax.experimental.pallas import tpu_sc as plsc

V = 1000000
D = 32
B = 4096
T = 200
NC, NS = 2, 16
NW = NC * NS
CHUNK = 128
NBUF = 8
AHEAD = 5

VP = 1 << 18
W1 = 16384
NLB = (V + W1 - 1) // W1


def _t1_body(x0, x1, x2, x3, o_ref):
    z = jnp.concatenate([x0[...], x1[...], x2[...], x3[...]], axis=0)
    o_ref[...] = z.T


def _t1_in_spec(p):
    return pl.BlockSpec(
        (D, W1), lambda i, p=p: (0, jnp.minimum(4 * i + p, NLB - 1)))


_t1 = pl.pallas_call(
    _t1_body,
    grid=(16,),
    in_specs=[_t1_in_spec(p) for p in range(4)],
    out_specs=pl.BlockSpec((W1, 128), lambda i: (i, 0)),
    out_shape=jax.ShapeDtypeStruct((W1 * 16, 128), jnp.float32),
)


T2_TB = 16


def _t2_body(x_ref, o_ref):
    for tt in range(T2_TB):
        z = x_ref[tt].T
        for u in range(4):
            o_ref[tt, :, 1024 * u:1024 * (u + 1)] = z[32 * u:32 * (u + 1), :]


_t2 = pl.pallas_call(
    _t2_body,
    grid=(T // T2_TB,),
    in_specs=[pl.BlockSpec((T2_TB, B // 4, 128), lambda i: (i, 0, 0))],
    out_specs=pl.BlockSpec((T2_TB, D, B), lambda i: (i, 0, 0)),
    out_shape=jax.ShapeDtypeStruct((T, D, B), jnp.float32),
)

_mesh = plsc.VectorSubcoreMesh(core_axis_name="c", subcore_axis_name="s")


@functools.partial(
    pl.kernel,
    mesh=_mesh,
    out_type=jax.ShapeDtypeStruct((B * T // 4, 4, D), jnp.float32),
    scratch_types=(
        [pltpu.VMEM((T, CHUNK), jnp.int32)]
        + [pltpu.VMEM((CHUNK, D), jnp.float32) for _ in range(NBUF)]
        + [pltpu.SemaphoreType.DMA for _ in range(2 * NBUF)]
    ),
    compiler_params=pltpu.CompilerParams(use_tc_tiling_on_sc=False),
)
def _gather_kernel(table_hbm, idx_hbm, out_hbm, idx_v, *scr):
    w = lax.axis_index("s") * NC + lax.axis_index("c")
    bufs = scr[:NBUF]
    gsems = scr[NBUF:2 * NBUF]
    wsems = scr[2 * NBUF:]
    pltpu.sync_copy(idx_hbm.at[w], idx_v)

    u = w // 8
    m0 = CHUNK * (w % 8)

    def out_slice(t):
        return out_hbm.at[pl.ds(t * (B // 4) + m0, CHUNK), u]

    def fire_gather(t, b):
        pltpu.async_copy(table_hbm.at[idx_v.at[t]], bufs[b], gsems[b])

    for b in range(AHEAD):
        fire_gather(b, b)

    def step(i, carry):
        for b in range(NBUF):
            t = NBUF * i + b

            @pl.when(t < T)
            def _():
                tn = t + AHEAD
                bn = (b + AHEAD) % NBUF

                @pl.when(tn < T)
                def _():
                    @pl.when(tn >= NBUF)
                    def _():
                        pltpu.make_async_copy(bufs[bn], out_slice(0),
                                              wsems[bn]).wait()
                    fire_gather(tn, bn)

                pltpu.make_async_copy(table_hbm.at[idx_v.at[t]], bufs[b],
                                      gsems[b]).wait()
                pltpu.async_copy(bufs[b], out_slice(t), wsems[b])
        return carry

    lax.fori_loop(0, (T + NBUF - 1) // NBUF, step, 0, unroll=False)

    for b in range(NBUF):
        pltpu.make_async_copy(bufs[b], out_slice(0), wsems[b]).wait()


def kernel(batch, weight):
    wt = weight.T
    table = _t1(wt, wt, wt, wt)
    table = table.reshape(W1 * 16 * 4, D)
    r = batch.astype(jnp.int32).reshape(NW, CHUNK, T).transpose(0, 2, 1)
    idx3 = (r & ~0xFFFF) | ((r & 0x3FFF) << 2) | ((r >> 14) & 3)
    flat3 = _gather_kernel(table, idx3)
    out_t = _t2(flat3.reshape(T, B // 4, 128))
    return out_t.transpose(2, 0, 1)
